# baseline (speedup 1.0000x reference)
import functools

import jax
import jax.numpy as jnp
from jax import lax
from jax.experimental import pallas as pl
from jax.experimental.pallas import tpu as pltpu
from jax.experimental.pallas import tpu_sc as plsc

NN = 100000
EE = 200000
UNITS = 32
BOND = 16

EPAD = 200704
EROWS = EPAD // 128
NHALF = 50000
ACC_ROWS = 50176
TRASH = 50048
ZROWS = ACC_ROWS // 16

G_ROWS = EROWS // 32
G_CH = 7
G_NCH = G_ROWS // G_CH

S_ROWS = EROWS // 16
S_CH = 2
S_NCH = S_ROWS // S_CH
S_NBUF = 3

BE = 4096
BN = 2000

_mesh = plsc.VectorSubcoreMesh(core_axis_name="c", subcore_axis_name="s")
_sc_params = pltpu.CompilerParams(use_tc_tiling_on_sc=False)


@functools.partial(
    pl.kernel,
    out_type=jax.ShapeDtypeStruct((EPAD, UNITS), jnp.float32),
    mesh=_mesh,
    compiler_params=_sc_params,
    scratch_types=[
        pltpu.VMEM((G_ROWS, 128), jnp.int32),
        [pltpu.VMEM((G_CH * 128, UNITS), jnp.float32)] * 4,
        [pltpu.SemaphoreType.DMA] * 4,
        [pltpu.SemaphoreType.DMA] * 4,
    ],
)
def _sc_gather(h_hbm, dst_hbm, out_hbm, idx_v, bufs, gsems, osems):
    cc = lax.axis_index("c")
    ss = lax.axis_index("s")
    wid = ss * 2 + cc
    base = wid * G_ROWS
    pltpu.sync_copy(dst_hbm.at[wid], idx_v)

    gcps = [None] * 4
    ocps = [None] * 4

    def fire(ch):
        b = ch % 4
        gcps[b] = [
            pltpu.async_copy(
                h_hbm.at[idx_v.at[ch * G_CH + j]],
                bufs[b].at[pl.ds(j * 128, 128)],
                gsems[b])
            for j in range(G_CH)]

    fire(0)
    fire(1)
    for ch in range(G_NCH):
        b = ch % 4
        nx = ch + 2
        if nx < G_NCH:
            nb = nx % 4
            if ocps[nb] is not None:
                ocps[nb].wait()
                ocps[nb] = None
            fire(nx)
        for cp in gcps[b]:
            cp.wait()
        ocps[b] = pltpu.async_copy(
            bufs[b],
            out_hbm.at[pl.ds((base + ch * G_CH) * 128, G_CH * 128)],
            osems[b])
    for b in range(4):
        if ocps[b] is not None:
            ocps[b].wait()


@functools.partial(
    pl.kernel,
    out_type=jax.ShapeDtypeStruct((NN, UNITS), jnp.float32),
    mesh=_mesh,
    compiler_params=_sc_params,
    scratch_types=[
        pltpu.VMEM_SHARED((ACC_ROWS, UNITS), jnp.float32),
        [pltpu.VMEM((S_CH, 128), jnp.int32)] * S_NBUF,
        [pltpu.VMEM((S_CH * 128, UNITS), jnp.float32)] * S_NBUF,
        [pltpu.SemaphoreType.DMA] * S_NBUF,
        [pltpu.SemaphoreType.DMA] * S_NBUF,
    ],
)
def _sc_scatter(msg_hbm, idx0_hbm, idx1_hbm, zer_hbm, agg_hbm,
                acc, idxbs, msgbs, lsems, asems):
    cc = lax.axis_index("c")
    ss = lax.axis_index("s")
    pltpu.sync_copy(zer_hbm, acc.at[pl.ds(ss * ZROWS, ZROWS)])
    base = ss * S_ROWS
    plsc.subcore_barrier()

    acps = [None] * S_NBUF

    def fire_load(ch):
        b = ch % S_NBUF

        @pl.when(cc == 0)
        def _():
            pltpu.async_copy(idx0_hbm.at[ss, ch], idxbs[b], lsems[b])

        @pl.when(cc == 1)
        def _():
            pltpu.async_copy(idx1_hbm.at[ss, ch], idxbs[b], lsems[b])

        pltpu.async_copy(
            msg_hbm.at[pl.ds((base + ch * S_CH) * 128, S_CH * 128)],
            msgbs[b], lsems[b])

    def wait_load(ch):
        b = ch % S_NBUF
        pltpu.make_async_copy(idx0_hbm.at[ss, ch], idxbs[b], lsems[b]).wait()
        pltpu.make_async_copy(
            msg_hbm.at[pl.ds((base + ch * S_CH) * 128, S_CH * 128)],
            msgbs[b], lsems[b]).wait()

    fire_load(0)
    fire_load(1)
    for ch in range(S_NCH):
        b = ch % S_NBUF
        wait_load(ch)
        acps[b] = [
            pltpu.async_copy(
                msgbs[b].at[pl.ds(j * 128, 128)],
                acc.at[idxbs[b].at[j]],
                asems[b], add=True)
            for j in range(S_CH)]
        nx = ch + 2
        if nx < S_NCH:
            nb = nx % S_NBUF
            if acps[nb] is not None:
                for cp in acps[nb]:
                    cp.wait()
                acps[nb] = None
            fire_load(nx)
    for b in range(S_NBUF):
        if acps[b] is not None:
            for cp in acps[b]:
                cp.wait()
    plsc.subcore_barrier()

    @pl.when(ss < 15)
    def _():
        pltpu.sync_copy(
            acc.at[pl.ds(ss * 3128, 3128)],
            agg_hbm.at[pl.ds(cc * NHALF + ss * 3128, 3128)])

    @pl.when(ss == 15)
    def _():
        pltpu.sync_copy(
            acc.at[pl.ds(15 * 3128, 3080)],
            agg_hbm.at[pl.ds(cc * NHALF + 15 * 3128, 3080)])


def _idx_kernel(src_ref, i0_ref, i1_ref):
    srcv = src_ref[...]
    lane = lax.broadcasted_iota(jnp.int32, (EROWS, 128), 1)
    dump = TRASH + lane
    ok0 = (srcv >= 0) & (srcv < NHALF)
    ok1 = (srcv >= NHALF) & (srcv < NN)
    i0_ref[...] = jnp.where(ok0, srcv, dump)
    i1_ref[...] = jnp.where(ok1, srcv - NHALF, dump)


def _edge_kernel(bond_ref, nbr_ref, r_ref, w2_ref, bt_ref, msg_ref):
    bond = bond_ref[...]
    nbr = nbr_ref[...]
    be = jnp.dot(bond, r_ref[...], preferred_element_type=jnp.float32)
    ne = jnp.concatenate([nbr] * BOND, axis=1)
    z = (be * ne).astype(jnp.bfloat16)
    msg_ref[...] = (
        jnp.dot(z, w2_ref[...], preferred_element_type=jnp.float32)
        + jnp.dot(nbr.astype(jnp.bfloat16),
                  bt_ref[...].astype(jnp.bfloat16),
                  preferred_element_type=jnp.float32))


def _gru_kernel(h_ref, agg_ref, wz, wr, wh, uz, ur, uh, b_ref, out_ref):
    h = h_ref[...]
    agg = agg_ref[...]
    b = b_ref[...]
    xz = jnp.dot(agg, wz[...], preferred_element_type=jnp.float32) + b[0:1, :]
    xr = jnp.dot(agg, wr[...], preferred_element_type=jnp.float32) + b[1:2, :]
    xh = jnp.dot(agg, wh[...], preferred_element_type=jnp.float32) + b[2:3, :]
    rz = jnp.dot(h, uz[...], preferred_element_type=jnp.float32) + b[3:4, :]
    rr = jnp.dot(h, ur[...], preferred_element_type=jnp.float32) + b[4:5, :]
    rh = jnp.dot(h, uh[...], preferred_element_type=jnp.float32) + b[5:6, :]
    z = jax.nn.sigmoid(xz + rz)
    r = jax.nn.sigmoid(xr + rr)
    hh = jnp.tanh(xh + r * rh)
    out_ref[...] = z * h + (1.0 - z) * hh


BE4 = BE // 4
EPAD4 = EPAD // 4
EB4 = EE // 4
NN4 = NN // 4
BN4 = 1000


def _edge_call(bond4, nbr4, rm4, w24, bt4):
    return pl.pallas_call(
        _edge_kernel,
        grid=(EPAD // BE,),
        in_specs=[
            pl.BlockSpec((BE4, 4 * BOND), lambda i: (i, 0)),
            pl.BlockSpec((BE4, 4 * UNITS), lambda i: (i, 0)),
            pl.BlockSpec((4 * BOND, 4 * BOND * UNITS), lambda i: (0, 0)),
            pl.BlockSpec((4 * BOND * UNITS, 4 * UNITS), lambda i: (0, 0)),
            pl.BlockSpec((4 * UNITS, 4 * UNITS), lambda i: (0, 0)),
        ],
        out_specs=pl.BlockSpec((BE4, 4 * UNITS), lambda i: (i, 0)),
        out_shape=jax.ShapeDtypeStruct((EPAD4, 4 * UNITS), jnp.float32),
    )(bond4, nbr4, rm4, w24, bt4)


def _gru_call(h4, agg4, wz, wr, wh, uz, ur, uh, bmat4):
    wspec = pl.BlockSpec((4 * UNITS, 4 * UNITS), lambda i: (0, 0))
    return pl.pallas_call(
        _gru_kernel,
        grid=(NN4 // BN4,),
        in_specs=[
            pl.BlockSpec((BN4, 4 * UNITS), lambda i: (i, 0)),
            pl.BlockSpec((BN4, 4 * UNITS), lambda i: (i, 0)),
            wspec, wspec, wspec, wspec, wspec, wspec,
            pl.BlockSpec((8, 4 * UNITS), lambda i: (0, 0)),
        ],
        out_specs=pl.BlockSpec((BN4, 4 * UNITS), lambda i: (i, 0)),
        out_shape=jax.ShapeDtypeStruct((NN4, 4 * UNITS), jnp.float32),
    )(h4, agg4, wz, wr, wh, uz, ur, uh, bmat4)


def kernel(atom_features, bond_features, pair_indices, edge_kernel,
           edge_bias, gru_kernel, gru_recurrent_kernel, gru_bias):
    h = atom_features
    src = pair_indices[:, 0]
    dst = pair_indices[:, 1]

    dst3 = jnp.concatenate(
        [dst, jnp.zeros((EPAD - EE,), jnp.int32)]).reshape(32, G_ROWS, 128)
    src2d = jnp.concatenate(
        [src, jnp.full((EPAD - EE,), -1, jnp.int32)]).reshape(EROWS, 128)
    bond4 = bond_features.astype(jnp.bfloat16).reshape(EB4, 4 * BOND)
    zer = jnp.zeros((ZROWS, UNITS), jnp.float32)

    eye4 = jnp.eye(4, dtype=jnp.float32)
    eye16 = jnp.eye(BOND, dtype=jnp.float32)
    w3 = edge_kernel.reshape(BOND, UNITS, UNITS)
    w2r = w3.transpose(0, 2, 1)
    bt = edge_bias.reshape(UNITS, UNITS).T
    rm4 = (eye4[:, None, None, :, None] * eye16[None, :, :, None, None]
           * jnp.ones((1, 1, 1, 1, UNITS), jnp.float32)
           ).reshape(4 * BOND, 4 * BOND * UNITS).astype(jnp.bfloat16)
    w24 = jnp.einsum('bvu,st->bsvtu', w2r, eye4).reshape(
        4 * BOND * UNITS, 4 * UNITS).astype(jnp.bfloat16)
    bt4 = jnp.kron(eye4, bt)

    wz, wr, wh = (gru_kernel[:, :UNITS], gru_kernel[:, UNITS:2 * UNITS],
                  gru_kernel[:, 2 * UNITS:])
    uz, ur, uh = (gru_recurrent_kernel[:, :UNITS],
                  gru_recurrent_kernel[:, UNITS:2 * UNITS],
                  gru_recurrent_kernel[:, 2 * UNITS:])
    wz4, wr4, wh4 = (jnp.kron(eye4, w) for w in (wz, wr, wh))
    uz4, ur4, uh4 = (jnp.kron(eye4, u) for u in (uz, ur, uh))
    bmat = jnp.concatenate(
        [gru_bias.reshape(6, UNITS), jnp.zeros((2, UNITS), jnp.float32)],
        axis=0)
    bmat4 = jnp.tile(bmat, (1, 4))

    idx0, idx1 = pl.pallas_call(
        _idx_kernel,
        out_shape=(jax.ShapeDtypeStruct((EROWS, 128), jnp.int32),
                   jax.ShapeDtypeStruct((EROWS, 128), jnp.int32)),
    )(src2d)
    idx0 = idx0.reshape(16, S_NCH, S_CH, 128)
    idx1 = idx1.reshape(16, S_NCH, S_CH, 128)

    for _ in range(2):
        nbr = _sc_gather(h, dst3)
        nbr4 = nbr.reshape(EPAD4, 4 * UNITS)
        msg4 = _edge_call(bond4, nbr4, rm4, w24, bt4)
        msg = msg4.reshape(EPAD, UNITS)
        agg = _sc_scatter(msg, idx0, idx1, zer)
        agg4 = agg.reshape(NN4, 4 * UNITS)
        h4 = h.reshape(NN4, 4 * UNITS)
        h4 = _gru_call(h4, agg4, wz4, wr4, wh4, uz4, ur4, uh4, bmat4)
        h = h4.reshape(NN, UNITS)
    return h


# speedup vs baseline: 1.0029x; 1.0029x over previous
import functools

import jax
import jax.numpy as jnp
from jax import lax
from jax.experimental import pallas as pl
from jax.experimental.pallas import tpu as pltpu
from jax.experimental.pallas import tpu_sc as plsc

NN = 100000
EE = 200000
UNITS = 32
BOND = 16

EPAD = 200704
EROWS = EPAD // 128
NHALF = 50000
ACC_ROWS = 50176
TRASH = 50048
ZROWS = ACC_ROWS // 16

G_ROWS = EROWS // 32
G_CH = 7
G_NCH = G_ROWS // G_CH

S_ROWS = EROWS // 16
S_CH = 2
S_NCH = S_ROWS // S_CH
S_NBUF = 3

BE = 4096
BN = 2000

_mesh = plsc.VectorSubcoreMesh(core_axis_name="c", subcore_axis_name="s")
_sc_params = pltpu.CompilerParams(use_tc_tiling_on_sc=False)


@functools.partial(
    pl.kernel,
    out_type=jax.ShapeDtypeStruct((EPAD, UNITS), jnp.float32),
    mesh=_mesh,
    compiler_params=_sc_params,
    scratch_types=[
        pltpu.VMEM((G_ROWS, 128), jnp.int32),
        [pltpu.VMEM((G_CH * 128, UNITS), jnp.float32)] * 4,
        [pltpu.SemaphoreType.DMA] * 4,
        [pltpu.SemaphoreType.DMA] * 4,
    ],
)
def _sc_gather(h_hbm, dst_hbm, out_hbm, idx_v, bufs, gsems, osems):
    cc = lax.axis_index("c")
    ss = lax.axis_index("s")
    wid = ss * 2 + cc
    base = wid * G_ROWS
    pltpu.sync_copy(dst_hbm.at[wid], idx_v)

    gcps = [None] * 4
    ocps = [None] * 4

    def fire(ch):
        b = ch % 4
        gcps[b] = [
            pltpu.async_copy(
                h_hbm.at[idx_v.at[ch * G_CH + j]],
                bufs[b].at[pl.ds(j * 128, 128)],
                gsems[b])
            for j in range(G_CH)]

    fire(0)
    fire(1)
    for ch in range(G_NCH):
        b = ch % 4
        nx = ch + 2
        if nx < G_NCH:
            nb = nx % 4
            if ocps[nb] is not None:
                ocps[nb].wait()
                ocps[nb] = None
            fire(nx)
        for cp in gcps[b]:
            cp.wait()
        ocps[b] = pltpu.async_copy(
            bufs[b],
            out_hbm.at[pl.ds((base + ch * G_CH) * 128, G_CH * 128)],
            osems[b])
    for b in range(4):
        if ocps[b] is not None:
            ocps[b].wait()


@functools.partial(
    pl.kernel,
    out_type=jax.ShapeDtypeStruct((NN, UNITS), jnp.float32),
    mesh=_mesh,
    compiler_params=_sc_params,
    scratch_types=[
        pltpu.VMEM_SHARED((ACC_ROWS, UNITS), jnp.float32),
        [pltpu.VMEM((S_CH, 128), jnp.int32)] * S_NBUF,
        [pltpu.VMEM((S_CH * 128, UNITS), jnp.float32)] * S_NBUF,
        [pltpu.SemaphoreType.DMA] * S_NBUF,
        [pltpu.SemaphoreType.DMA] * S_NBUF,
    ],
)
def _sc_scatter(msg_hbm, idx0_hbm, idx1_hbm, zer_hbm, agg_hbm,
                acc, idxbs, msgbs, lsems, asems):
    cc = lax.axis_index("c")
    ss = lax.axis_index("s")
    pltpu.sync_copy(zer_hbm, acc.at[pl.ds(ss * ZROWS, ZROWS)])
    base = ss * S_ROWS
    plsc.subcore_barrier()

    acps = [None] * S_NBUF

    def fire_load(ch):
        b = ch % S_NBUF

        @pl.when(cc == 0)
        def _():
            pltpu.async_copy(idx0_hbm.at[ss, ch], idxbs[b], lsems[b])

        @pl.when(cc == 1)
        def _():
            pltpu.async_copy(idx1_hbm.at[ss, ch], idxbs[b], lsems[b])

        pltpu.async_copy(
            msg_hbm.at[pl.ds((base + ch * S_CH) * 128, S_CH * 128)],
            msgbs[b], lsems[b])

    def wait_load(ch):
        b = ch % S_NBUF
        pltpu.make_async_copy(idx0_hbm.at[ss, ch], idxbs[b], lsems[b]).wait()
        pltpu.make_async_copy(
            msg_hbm.at[pl.ds((base + ch * S_CH) * 128, S_CH * 128)],
            msgbs[b], lsems[b]).wait()

    fire_load(0)
    fire_load(1)
    for ch in range(S_NCH):
        b = ch % S_NBUF
        wait_load(ch)
        acps[b] = [
            pltpu.async_copy(
                msgbs[b].at[pl.ds(j * 128, 128)],
                acc.at[idxbs[b].at[j]],
                asems[b], add=True)
            for j in range(S_CH)]
        nx = ch + 2
        if nx < S_NCH:
            nb = nx % S_NBUF
            if acps[nb] is not None:
                for cp in acps[nb]:
                    cp.wait()
                acps[nb] = None
            fire_load(nx)
    for b in range(S_NBUF):
        if acps[b] is not None:
            for cp in acps[b]:
                cp.wait()
    plsc.subcore_barrier()

    @pl.when(ss < 15)
    def _():
        pltpu.sync_copy(
            acc.at[pl.ds(ss * 3128, 3128)],
            agg_hbm.at[pl.ds(cc * NHALF + ss * 3128, 3128)])

    @pl.when(ss == 15)
    def _():
        pltpu.sync_copy(
            acc.at[pl.ds(15 * 3128, 3080)],
            agg_hbm.at[pl.ds(cc * NHALF + 15 * 3128, 3080)])


def _idx_kernel(src_ref, i0_ref, i1_ref):
    srcv = src_ref[...]
    lane = lax.broadcasted_iota(jnp.int32, (EROWS, 128), 1)
    dump = TRASH + lane
    ok0 = (srcv >= 0) & (srcv < NHALF)
    ok1 = (srcv >= NHALF) & (srcv < NN)
    i0_ref[...] = jnp.where(ok0, srcv, dump)
    i1_ref[...] = jnp.where(ok1, srcv - NHALF, dump)


def _edge_kernel(bond_ref, nbr_ref, r_ref, w2_ref, bt_ref, msg_ref):
    bond = bond_ref[...]
    nbr = nbr_ref[...]
    be = jnp.dot(bond, r_ref[...], preferred_element_type=jnp.float32)
    ne = jnp.concatenate([nbr] * BOND, axis=1)
    msg_ref[...] = (
        jnp.dot(be * ne, w2_ref[...], preferred_element_type=jnp.float32)
        + jnp.dot(nbr.astype(jnp.bfloat16),
                  bt_ref[...].astype(jnp.bfloat16),
                  preferred_element_type=jnp.float32))


def _gru_kernel(h_ref, agg_ref, wz, wr, wh, uz, ur, uh, b_ref, out_ref):
    h = h_ref[...]
    agg = agg_ref[...]
    b = b_ref[...]
    xz = jnp.dot(agg, wz[...], preferred_element_type=jnp.float32) + b[0:1, :]
    xr = jnp.dot(agg, wr[...], preferred_element_type=jnp.float32) + b[1:2, :]
    xh = jnp.dot(agg, wh[...], preferred_element_type=jnp.float32) + b[2:3, :]
    rz = jnp.dot(h, uz[...], preferred_element_type=jnp.float32) + b[3:4, :]
    rr = jnp.dot(h, ur[...], preferred_element_type=jnp.float32) + b[4:5, :]
    rh = jnp.dot(h, uh[...], preferred_element_type=jnp.float32) + b[5:6, :]
    z = jax.nn.sigmoid(xz + rz)
    r = jax.nn.sigmoid(xr + rr)
    hh = jnp.tanh(xh + r * rh)
    out_ref[...] = z * h + (1.0 - z) * hh


BE4 = BE // 4
EPAD4 = EPAD // 4
EB4 = EE // 4
NN4 = NN // 4
BN4 = 1000


def _edge_call(bond4, nbr4, rm4, w24, bt4):
    return pl.pallas_call(
        _edge_kernel,
        grid=(EPAD // BE,),
        in_specs=[
            pl.BlockSpec((BE4, 4 * BOND), lambda i: (i, 0)),
            pl.BlockSpec((BE4, 4 * UNITS), lambda i: (i, 0)),
            pl.BlockSpec((4 * BOND, 4 * BOND * UNITS), lambda i: (0, 0)),
            pl.BlockSpec((4 * BOND * UNITS, 4 * UNITS), lambda i: (0, 0)),
            pl.BlockSpec((4 * UNITS, 4 * UNITS), lambda i: (0, 0)),
        ],
        out_specs=pl.BlockSpec((BE4, 4 * UNITS), lambda i: (i, 0)),
        out_shape=jax.ShapeDtypeStruct((EPAD4, 4 * UNITS), jnp.float32),
    )(bond4, nbr4, rm4, w24, bt4)


def _gru_call(h4, agg4, wz, wr, wh, uz, ur, uh, bmat4):
    wspec = pl.BlockSpec((4 * UNITS, 4 * UNITS), lambda i: (0, 0))
    return pl.pallas_call(
        _gru_kernel,
        grid=(NN4 // BN4,),
        in_specs=[
            pl.BlockSpec((BN4, 4 * UNITS), lambda i: (i, 0)),
            pl.BlockSpec((BN4, 4 * UNITS), lambda i: (i, 0)),
            wspec, wspec, wspec, wspec, wspec, wspec,
            pl.BlockSpec((8, 4 * UNITS), lambda i: (0, 0)),
        ],
        out_specs=pl.BlockSpec((BN4, 4 * UNITS), lambda i: (i, 0)),
        out_shape=jax.ShapeDtypeStruct((NN4, 4 * UNITS), jnp.float32),
    )(h4, agg4, wz, wr, wh, uz, ur, uh, bmat4)


def kernel(atom_features, bond_features, pair_indices, edge_kernel,
           edge_bias, gru_kernel, gru_recurrent_kernel, gru_bias):
    h = atom_features
    src = pair_indices[:, 0]
    dst = pair_indices[:, 1]

    dst3 = jnp.concatenate(
        [dst, jnp.zeros((EPAD - EE,), jnp.int32)]).reshape(32, G_ROWS, 128)
    src2d = jnp.concatenate(
        [src, jnp.full((EPAD - EE,), -1, jnp.int32)]).reshape(EROWS, 128)
    bond4 = bond_features.astype(jnp.bfloat16).reshape(EB4, 4 * BOND)
    zer = jnp.zeros((ZROWS, UNITS), jnp.float32)

    eye4 = jnp.eye(4, dtype=jnp.float32)
    eye16 = jnp.eye(BOND, dtype=jnp.float32)
    w3 = edge_kernel.reshape(BOND, UNITS, UNITS)
    w2r = w3.transpose(0, 2, 1)
    bt = edge_bias.reshape(UNITS, UNITS).T
    rm4 = (eye4[:, None, None, :, None] * eye16[None, :, :, None, None]
           * jnp.ones((1, 1, 1, 1, UNITS), jnp.float32)
           ).reshape(4 * BOND, 4 * BOND * UNITS).astype(jnp.bfloat16)
    w24 = jnp.einsum('bvu,st->bsvtu', w2r, eye4).reshape(
        4 * BOND * UNITS, 4 * UNITS)
    bt4 = jnp.kron(eye4, bt)

    wz, wr, wh = (gru_kernel[:, :UNITS], gru_kernel[:, UNITS:2 * UNITS],
                  gru_kernel[:, 2 * UNITS:])
    uz, ur, uh = (gru_recurrent_kernel[:, :UNITS],
                  gru_recurrent_kernel[:, UNITS:2 * UNITS],
                  gru_recurrent_kernel[:, 2 * UNITS:])
    wz4, wr4, wh4 = (jnp.kron(eye4, w) for w in (wz, wr, wh))
    uz4, ur4, uh4 = (jnp.kron(eye4, u) for u in (uz, ur, uh))
    bmat = jnp.concatenate(
        [gru_bias.reshape(6, UNITS), jnp.zeros((2, UNITS), jnp.float32)],
        axis=0)
    bmat4 = jnp.tile(bmat, (1, 4))

    idx0, idx1 = pl.pallas_call(
        _idx_kernel,
        out_shape=(jax.ShapeDtypeStruct((EROWS, 128), jnp.int32),
                   jax.ShapeDtypeStruct((EROWS, 128), jnp.int32)),
    )(src2d)
    idx0 = idx0.reshape(16, S_NCH, S_CH, 128)
    idx1 = idx1.reshape(16, S_NCH, S_CH, 128)

    for _ in range(2):
        nbr = _sc_gather(h, dst3)
        nbr4 = nbr.reshape(EPAD4, 4 * UNITS)
        msg4 = _edge_call(bond4, nbr4, rm4, w24, bt4)
        msg = msg4.reshape(EPAD, UNITS)
        agg = _sc_scatter(msg, idx0, idx1, zer)
        agg4 = agg.reshape(NN4, 4 * UNITS)
        h4 = h.reshape(NN4, 4 * UNITS)
        h4 = _gru_call(h4, agg4, wz4, wr4, wh4, uz4, ur4, uh4, bmat4)
        h = h4.reshape(NN, UNITS)
    return h


# speedup vs baseline: 1.0260x; 1.0230x over previous
import functools

import jax
import jax.numpy as jnp
from jax import lax
from jax.experimental import pallas as pl
from jax.experimental.pallas import tpu as pltpu
from jax.experimental.pallas import tpu_sc as plsc

NN = 100000
EE = 200000
UNITS = 32
BOND = 16

EPAD = 200704
EROWS = EPAD // 128
NHALF = 50000
ACC_ROWS = 50176
TRASH = 50048
ZROWS = ACC_ROWS // 16

G_ROWS = EROWS // 32
G_CH = 7
G_NCH = G_ROWS // G_CH

S_ROWS = EROWS // 16
S_CH = 2
S_NCH = S_ROWS // S_CH
S_NBUF = 3

BE = 4096
BN = 2000

_mesh = plsc.VectorSubcoreMesh(core_axis_name="c", subcore_axis_name="s")
_sc_params = pltpu.CompilerParams(use_tc_tiling_on_sc=False)


@functools.partial(
    pl.kernel,
    out_type=jax.ShapeDtypeStruct((EPAD, UNITS), jnp.float32),
    mesh=_mesh,
    compiler_params=_sc_params,
    scratch_types=[
        pltpu.VMEM((G_ROWS, 128), jnp.int32),
        [pltpu.VMEM((G_CH * 128, UNITS), jnp.float32)] * 4,
        [pltpu.SemaphoreType.DMA] * 4,
        [pltpu.SemaphoreType.DMA] * 4,
    ],
)
def _sc_gather(h_hbm, dst_hbm, out_hbm, idx_v, bufs, gsems, osems):
    cc = lax.axis_index("c")
    ss = lax.axis_index("s")
    wid = ss * 2 + cc
    base = wid * G_ROWS
    pltpu.sync_copy(dst_hbm.at[wid], idx_v)

    gcps = [None] * 4
    ocps = [None] * 4

    def fire(ch):
        b = ch % 4
        gcps[b] = [
            pltpu.async_copy(
                h_hbm.at[idx_v.at[ch * G_CH + j]],
                bufs[b].at[pl.ds(j * 128, 128)],
                gsems[b])
            for j in range(G_CH)]

    fire(0)
    fire(1)
    for ch in range(G_NCH):
        b = ch % 4
        nx = ch + 2
        if nx < G_NCH:
            nb = nx % 4
            if ocps[nb] is not None:
                ocps[nb].wait()
                ocps[nb] = None
            fire(nx)
        for cp in gcps[b]:
            cp.wait()
        ocps[b] = pltpu.async_copy(
            bufs[b],
            out_hbm.at[pl.ds((base + ch * G_CH) * 128, G_CH * 128)],
            osems[b])
    for b in range(4):
        if ocps[b] is not None:
            ocps[b].wait()


@functools.partial(
    pl.kernel,
    out_type=jax.ShapeDtypeStruct((NN, UNITS), jnp.float32),
    mesh=_mesh,
    compiler_params=_sc_params,
    scratch_types=[
        pltpu.VMEM_SHARED((ACC_ROWS, UNITS), jnp.float32),
        [pltpu.VMEM((S_CH, 128), jnp.int32)] * S_NBUF,
        [pltpu.VMEM((S_CH * 128, UNITS), jnp.float32)] * S_NBUF,
        [pltpu.SemaphoreType.DMA] * S_NBUF,
        [pltpu.SemaphoreType.DMA] * S_NBUF,
    ],
)
def _sc_scatter(msg_hbm, idx0_hbm, idx1_hbm, zer_hbm, agg_hbm,
                acc, idxbs, msgbs, lsems, asems):
    cc = lax.axis_index("c")
    ss = lax.axis_index("s")
    pltpu.sync_copy(zer_hbm, acc.at[pl.ds(ss * ZROWS, ZROWS)])
    base = ss * S_ROWS
    plsc.subcore_barrier()

    acps = [None] * S_NBUF

    def fire_load(ch):
        b = ch % S_NBUF

        @pl.when(cc == 0)
        def _():
            pltpu.async_copy(idx0_hbm.at[ss, ch], idxbs[b], lsems[b])

        @pl.when(cc == 1)
        def _():
            pltpu.async_copy(idx1_hbm.at[ss, ch], idxbs[b], lsems[b])

        pltpu.async_copy(
            msg_hbm.at[pl.ds((base + ch * S_CH) * 128, S_CH * 128)],
            msgbs[b], lsems[b])

    def wait_load(ch):
        b = ch % S_NBUF
        pltpu.make_async_copy(idx0_hbm.at[ss, ch], idxbs[b], lsems[b]).wait()
        pltpu.make_async_copy(
            msg_hbm.at[pl.ds((base + ch * S_CH) * 128, S_CH * 128)],
            msgbs[b], lsems[b]).wait()

    fire_load(0)
    fire_load(1)
    for ch in range(S_NCH):
        b = ch % S_NBUF
        wait_load(ch)
        acps[b] = [
            pltpu.async_copy(
                msgbs[b].at[pl.ds(j * 128, 128)],
                acc.at[idxbs[b].at[j]],
                asems[b], add=True)
            for j in range(S_CH)]
        nx = ch + 2
        if nx < S_NCH:
            nb = nx % S_NBUF
            if acps[nb] is not None:
                for cp in acps[nb]:
                    cp.wait()
                acps[nb] = None
            fire_load(nx)
    for b in range(S_NBUF):
        if acps[b] is not None:
            for cp in acps[b]:
                cp.wait()
    plsc.subcore_barrier()

    @pl.when(ss < 15)
    def _():
        pltpu.sync_copy(
            acc.at[pl.ds(ss * 3128, 3128)],
            agg_hbm.at[pl.ds(cc * NHALF + ss * 3128, 3128)])

    @pl.when(ss == 15)
    def _():
        pltpu.sync_copy(
            acc.at[pl.ds(15 * 3128, 3080)],
            agg_hbm.at[pl.ds(cc * NHALF + 15 * 3128, 3080)])


def _idx_kernel(src_ref, i0_ref, i1_ref):
    srcv = src_ref[...]
    lane = lax.broadcasted_iota(jnp.int32, (EROWS, 128), 1)
    dump = TRASH + lane
    ok0 = (srcv >= 0) & (srcv < NHALF)
    ok1 = (srcv >= NHALF) & (srcv < NN)
    i0_ref[...] = jnp.where(ok0, srcv, dump)
    i1_ref[...] = jnp.where(ok1, srcv - NHALF, dump)


def _edge_kernel(bond_ref, nbr_ref, r_ref, w2_ref, bt_ref, msg_ref):
    bond = bond_ref[...]
    nbr = nbr_ref[...]
    be = jnp.dot(bond, r_ref[...], preferred_element_type=jnp.float32)
    ne = jnp.concatenate([nbr] * BOND, axis=1)
    msg_ref[...] = (
        jnp.dot(be * ne, w2_ref[...], preferred_element_type=jnp.float32)
        + jnp.dot(nbr.astype(jnp.bfloat16),
                  bt_ref[...].astype(jnp.bfloat16),
                  preferred_element_type=jnp.float32))


def _gru_kernel(h_ref, agg_ref, wz, wr, wh, uz, ur, uh, b_ref, out_ref):
    h = h_ref[...]
    agg = agg_ref[...]
    b = b_ref[...]
    xz = jnp.dot(agg, wz[...], preferred_element_type=jnp.float32) + b[0:1, :]
    xr = jnp.dot(agg, wr[...], preferred_element_type=jnp.float32) + b[1:2, :]
    xh = jnp.dot(agg, wh[...], preferred_element_type=jnp.float32) + b[2:3, :]
    rz = jnp.dot(h, uz[...], preferred_element_type=jnp.float32) + b[3:4, :]
    rr = jnp.dot(h, ur[...], preferred_element_type=jnp.float32) + b[4:5, :]
    rh = jnp.dot(h, uh[...], preferred_element_type=jnp.float32) + b[5:6, :]
    z = jax.nn.sigmoid(xz + rz)
    r = jax.nn.sigmoid(xr + rr)
    hh = jnp.tanh(xh + r * rh)
    out_ref[...] = z * h + (1.0 - z) * hh


BE4 = BE // 4
EPAD4 = EPAD // 4
EB4 = EE // 4
NN4 = NN // 4
BN4 = 5000


def _edge_call(bond4, nbr4, rm4, w24, bt4):
    return pl.pallas_call(
        _edge_kernel,
        grid=(EPAD // BE,),
        in_specs=[
            pl.BlockSpec((BE4, 4 * BOND), lambda i: (i, 0)),
            pl.BlockSpec((BE4, 4 * UNITS), lambda i: (i, 0)),
            pl.BlockSpec((4 * BOND, 4 * BOND * UNITS), lambda i: (0, 0)),
            pl.BlockSpec((4 * BOND * UNITS, 4 * UNITS), lambda i: (0, 0)),
            pl.BlockSpec((4 * UNITS, 4 * UNITS), lambda i: (0, 0)),
        ],
        out_specs=pl.BlockSpec((BE4, 4 * UNITS), lambda i: (i, 0)),
        out_shape=jax.ShapeDtypeStruct((EPAD4, 4 * UNITS), jnp.float32),
    )(bond4, nbr4, rm4, w24, bt4)


def _gru_call(h4, agg4, wz, wr, wh, uz, ur, uh, bmat4):
    wspec = pl.BlockSpec((4 * UNITS, 4 * UNITS), lambda i: (0, 0))
    return pl.pallas_call(
        _gru_kernel,
        grid=(NN4 // BN4,),
        in_specs=[
            pl.BlockSpec((BN4, 4 * UNITS), lambda i: (i, 0)),
            pl.BlockSpec((BN4, 4 * UNITS), lambda i: (i, 0)),
            wspec, wspec, wspec, wspec, wspec, wspec,
            pl.BlockSpec((8, 4 * UNITS), lambda i: (0, 0)),
        ],
        out_specs=pl.BlockSpec((BN4, 4 * UNITS), lambda i: (i, 0)),
        out_shape=jax.ShapeDtypeStruct((NN4, 4 * UNITS), jnp.float32),
    )(h4, agg4, wz, wr, wh, uz, ur, uh, bmat4)


def kernel(atom_features, bond_features, pair_indices, edge_kernel,
           edge_bias, gru_kernel, gru_recurrent_kernel, gru_bias):
    h = atom_features
    src = pair_indices[:, 0]
    dst = pair_indices[:, 1]

    dst3 = jnp.concatenate(
        [dst, jnp.zeros((EPAD - EE,), jnp.int32)]).reshape(32, G_ROWS, 128)
    src2d = jnp.concatenate(
        [src, jnp.full((EPAD - EE,), -1, jnp.int32)]).reshape(EROWS, 128)
    bond4 = bond_features.astype(jnp.bfloat16).reshape(EB4, 4 * BOND)
    zer = jnp.zeros((ZROWS, UNITS), jnp.float32)

    eye4 = jnp.eye(4, dtype=jnp.float32)
    eye16 = jnp.eye(BOND, dtype=jnp.float32)
    w3 = edge_kernel.reshape(BOND, UNITS, UNITS)
    w2r = w3.transpose(0, 2, 1)
    bt = edge_bias.reshape(UNITS, UNITS).T
    rm4 = (eye4[:, None, None, :, None] * eye16[None, :, :, None, None]
           * jnp.ones((1, 1, 1, 1, UNITS), jnp.float32)
           ).reshape(4 * BOND, 4 * BOND * UNITS).astype(jnp.bfloat16)
    w24 = jnp.einsum('bvu,st->bsvtu', w2r, eye4).reshape(
        4 * BOND * UNITS, 4 * UNITS)
    bt4 = jnp.kron(eye4, bt)

    wz, wr, wh = (gru_kernel[:, :UNITS], gru_kernel[:, UNITS:2 * UNITS],
                  gru_kernel[:, 2 * UNITS:])
    uz, ur, uh = (gru_recurrent_kernel[:, :UNITS],
                  gru_recurrent_kernel[:, UNITS:2 * UNITS],
                  gru_recurrent_kernel[:, 2 * UNITS:])
    wz4, wr4, wh4 = (jnp.kron(eye4, w) for w in (wz, wr, wh))
    uz4, ur4, uh4 = (jnp.kron(eye4, u) for u in (uz, ur, uh))
    bmat = jnp.concatenate(
        [gru_bias.reshape(6, UNITS), jnp.zeros((2, UNITS), jnp.float32)],
        axis=0)
    bmat4 = jnp.tile(bmat, (1, 4))

    idx0, idx1 = pl.pallas_call(
        _idx_kernel,
        out_shape=(jax.ShapeDtypeStruct((EROWS, 128), jnp.int32),
                   jax.ShapeDtypeStruct((EROWS, 128), jnp.int32)),
    )(src2d)
    idx0 = idx0.reshape(16, S_NCH, S_CH, 128)
    idx1 = idx1.reshape(16, S_NCH, S_CH, 128)

    for _ in range(2):
        nbr = _sc_gather(h, dst3)
        nbr4 = nbr.reshape(EPAD4, 4 * UNITS)
        msg4 = _edge_call(bond4, nbr4, rm4, w24, bt4)
        msg = msg4.reshape(EPAD, UNITS)
        agg = _sc_scatter(msg, idx0, idx1, zer)
        agg4 = agg.reshape(NN4, 4 * UNITS)
        h4 = h.reshape(NN4, 4 * UNITS)
        h4 = _gru_call(h4, agg4, wz4, wr4, wh4, uz4, ur4, uh4, bmat4)
        h = h4.reshape(NN, UNITS)
    return h


# speedup vs baseline: 1.0431x; 1.0166x over previous
import functools

import jax
import jax.numpy as jnp
from jax import lax
from jax.experimental import pallas as pl
from jax.experimental.pallas import tpu as pltpu
from jax.experimental.pallas import tpu_sc as plsc

NN = 100000
EE = 200000
UNITS = 32
BOND = 16

EPAD = 200704
EROWS = EPAD // 128
NHALF = 50000
ACC_ROWS = 50176
TRASH = 50048
ZROWS = ACC_ROWS // 16

G_ROWS = EROWS // 32
G_CH = 7
G_NCH = G_ROWS // G_CH

S_ROWS = EROWS // 16
S_CH = 2
S_NCH = S_ROWS // S_CH
S_NBUF = 3

BE = 6272
BN = 2000

_mesh = plsc.VectorSubcoreMesh(core_axis_name="c", subcore_axis_name="s")
_sc_params = pltpu.CompilerParams(use_tc_tiling_on_sc=False)


@functools.partial(
    pl.kernel,
    out_type=jax.ShapeDtypeStruct((EPAD, UNITS), jnp.float32),
    mesh=_mesh,
    compiler_params=_sc_params,
    scratch_types=[
        pltpu.VMEM((G_ROWS, 128), jnp.int32),
        [pltpu.VMEM((G_CH * 128, UNITS), jnp.float32)] * 4,
        [pltpu.SemaphoreType.DMA] * 4,
        [pltpu.SemaphoreType.DMA] * 4,
    ],
)
def _sc_gather(h_hbm, dst_hbm, out_hbm, idx_v, bufs, gsems, osems):
    cc = lax.axis_index("c")
    ss = lax.axis_index("s")
    wid = ss * 2 + cc
    base = wid * G_ROWS
    pltpu.sync_copy(dst_hbm.at[wid], idx_v)

    gcps = [None] * 4
    ocps = [None] * 4

    def fire(ch):
        b = ch % 4
        gcps[b] = [
            pltpu.async_copy(
                h_hbm.at[idx_v.at[ch * G_CH + j]],
                bufs[b].at[pl.ds(j * 128, 128)],
                gsems[b])
            for j in range(G_CH)]

    fire(0)
    fire(1)
    for ch in range(G_NCH):
        b = ch % 4
        nx = ch + 2
        if nx < G_NCH:
            nb = nx % 4
            if ocps[nb] is not None:
                ocps[nb].wait()
                ocps[nb] = None
            fire(nx)
        for cp in gcps[b]:
            cp.wait()
        ocps[b] = pltpu.async_copy(
            bufs[b],
            out_hbm.at[pl.ds((base + ch * G_CH) * 128, G_CH * 128)],
            osems[b])
    for b in range(4):
        if ocps[b] is not None:
            ocps[b].wait()


@functools.partial(
    pl.kernel,
    out_type=jax.ShapeDtypeStruct((NN, UNITS), jnp.float32),
    mesh=_mesh,
    compiler_params=_sc_params,
    scratch_types=[
        pltpu.VMEM_SHARED((ACC_ROWS, UNITS), jnp.float32),
        [pltpu.VMEM((S_CH, 128), jnp.int32)] * S_NBUF,
        [pltpu.VMEM((S_CH * 128, UNITS), jnp.float32)] * S_NBUF,
        [pltpu.SemaphoreType.DMA] * S_NBUF,
        [pltpu.SemaphoreType.DMA] * S_NBUF,
    ],
)
def _sc_scatter(msg_hbm, idx0_hbm, idx1_hbm, zer_hbm, agg_hbm,
                acc, idxbs, msgbs, lsems, asems):
    cc = lax.axis_index("c")
    ss = lax.axis_index("s")
    pltpu.sync_copy(zer_hbm, acc.at[pl.ds(ss * ZROWS, ZROWS)])
    base = ss * S_ROWS
    plsc.subcore_barrier()

    acps = [None] * S_NBUF

    def fire_load(ch):
        b = ch % S_NBUF

        @pl.when(cc == 0)
        def _():
            pltpu.async_copy(idx0_hbm.at[ss, ch], idxbs[b], lsems[b])

        @pl.when(cc == 1)
        def _():
            pltpu.async_copy(idx1_hbm.at[ss, ch], idxbs[b], lsems[b])

        pltpu.async_copy(
            msg_hbm.at[pl.ds((base + ch * S_CH) * 128, S_CH * 128)],
            msgbs[b], lsems[b])

    def wait_load(ch):
        b = ch % S_NBUF
        pltpu.make_async_copy(idx0_hbm.at[ss, ch], idxbs[b], lsems[b]).wait()
        pltpu.make_async_copy(
            msg_hbm.at[pl.ds((base + ch * S_CH) * 128, S_CH * 128)],
            msgbs[b], lsems[b]).wait()

    fire_load(0)
    fire_load(1)
    for ch in range(S_NCH):
        b = ch % S_NBUF
        wait_load(ch)
        acps[b] = [
            pltpu.async_copy(
                msgbs[b].at[pl.ds(j * 128, 128)],
                acc.at[idxbs[b].at[j]],
                asems[b], add=True)
            for j in range(S_CH)]
        nx = ch + 2
        if nx < S_NCH:
            nb = nx % S_NBUF
            if acps[nb] is not None:
                for cp in acps[nb]:
                    cp.wait()
                acps[nb] = None
            fire_load(nx)
    for b in range(S_NBUF):
        if acps[b] is not None:
            for cp in acps[b]:
                cp.wait()
    plsc.subcore_barrier()

    @pl.when(ss < 15)
    def _():
        pltpu.sync_copy(
            acc.at[pl.ds(ss * 3128, 3128)],
            agg_hbm.at[pl.ds(cc * NHALF + ss * 3128, 3128)])

    @pl.when(ss == 15)
    def _():
        pltpu.sync_copy(
            acc.at[pl.ds(15 * 3128, 3080)],
            agg_hbm.at[pl.ds(cc * NHALF + 15 * 3128, 3080)])


def _idx_kernel(src_ref, i0_ref, i1_ref):
    srcv = src_ref[...]
    lane = lax.broadcasted_iota(jnp.int32, (EROWS, 128), 1)
    dump = TRASH + lane
    ok0 = (srcv >= 0) & (srcv < NHALF)
    ok1 = (srcv >= NHALF) & (srcv < NN)
    i0_ref[...] = jnp.where(ok0, srcv, dump)
    i1_ref[...] = jnp.where(ok1, srcv - NHALF, dump)


def _edge_kernel(bond_ref, nbr_ref, r_ref, w2_ref, bt_ref, msg_ref):
    bond = bond_ref[...]
    nbr = nbr_ref[...]
    be = jnp.dot(bond, r_ref[...], preferred_element_type=jnp.float32)
    ne = jnp.concatenate([nbr] * BOND, axis=1)
    msg_ref[...] = (
        jnp.dot(be * ne, w2_ref[...], preferred_element_type=jnp.float32)
        + jnp.dot(nbr.astype(jnp.bfloat16),
                  bt_ref[...].astype(jnp.bfloat16),
                  preferred_element_type=jnp.float32))


def _gru_kernel(h_ref, agg_ref, wz, wr, wh, uz, ur, uh, b_ref, out_ref):
    h = h_ref[...]
    agg = agg_ref[...]
    b = b_ref[...]
    xz = jnp.dot(agg, wz[...], preferred_element_type=jnp.float32) + b[0:1, :]
    xr = jnp.dot(agg, wr[...], preferred_element_type=jnp.float32) + b[1:2, :]
    xh = jnp.dot(agg, wh[...], preferred_element_type=jnp.float32) + b[2:3, :]
    rz = jnp.dot(h, uz[...], preferred_element_type=jnp.float32) + b[3:4, :]
    rr = jnp.dot(h, ur[...], preferred_element_type=jnp.float32) + b[4:5, :]
    rh = jnp.dot(h, uh[...], preferred_element_type=jnp.float32) + b[5:6, :]
    z = jax.nn.sigmoid(xz + rz)
    r = jax.nn.sigmoid(xr + rr)
    hh = jnp.tanh(xh + r * rh)
    out_ref[...] = z * h + (1.0 - z) * hh


BE4 = BE // 4
EPAD4 = EPAD // 4
EB4 = EE // 4
NN4 = NN // 4
BN4 = 5000


def _edge_call(bond4, nbr4, rm4, w24, bt4):
    return pl.pallas_call(
        _edge_kernel,
        grid=(EPAD // BE,),
        in_specs=[
            pl.BlockSpec((BE4, 4 * BOND), lambda i: (i, 0)),
            pl.BlockSpec((BE4, 4 * UNITS), lambda i: (i, 0)),
            pl.BlockSpec((4 * BOND, 4 * BOND * UNITS), lambda i: (0, 0)),
            pl.BlockSpec((4 * BOND * UNITS, 4 * UNITS), lambda i: (0, 0)),
            pl.BlockSpec((4 * UNITS, 4 * UNITS), lambda i: (0, 0)),
        ],
        out_specs=pl.BlockSpec((BE4, 4 * UNITS), lambda i: (i, 0)),
        out_shape=jax.ShapeDtypeStruct((EPAD4, 4 * UNITS), jnp.float32),
    )(bond4, nbr4, rm4, w24, bt4)


def _gru_call(h4, agg4, wz, wr, wh, uz, ur, uh, bmat4):
    wspec = pl.BlockSpec((4 * UNITS, 4 * UNITS), lambda i: (0, 0))
    return pl.pallas_call(
        _gru_kernel,
        grid=(NN4 // BN4,),
        in_specs=[
            pl.BlockSpec((BN4, 4 * UNITS), lambda i: (i, 0)),
            pl.BlockSpec((BN4, 4 * UNITS), lambda i: (i, 0)),
            wspec, wspec, wspec, wspec, wspec, wspec,
            pl.BlockSpec((8, 4 * UNITS), lambda i: (0, 0)),
        ],
        out_specs=pl.BlockSpec((BN4, 4 * UNITS), lambda i: (i, 0)),
        out_shape=jax.ShapeDtypeStruct((NN4, 4 * UNITS), jnp.float32),
    )(h4, agg4, wz, wr, wh, uz, ur, uh, bmat4)


def kernel(atom_features, bond_features, pair_indices, edge_kernel,
           edge_bias, gru_kernel, gru_recurrent_kernel, gru_bias):
    h = atom_features
    src = pair_indices[:, 0]
    dst = pair_indices[:, 1]

    dst3 = jnp.concatenate(
        [dst, jnp.zeros((EPAD - EE,), jnp.int32)]).reshape(32, G_ROWS, 128)
    src2d = jnp.concatenate(
        [src, jnp.full((EPAD - EE,), -1, jnp.int32)]).reshape(EROWS, 128)
    bond4 = bond_features.astype(jnp.bfloat16).reshape(EB4, 4 * BOND)
    zer = jnp.zeros((ZROWS, UNITS), jnp.float32)

    eye4 = jnp.eye(4, dtype=jnp.float32)
    eye16 = jnp.eye(BOND, dtype=jnp.float32)
    w3 = edge_kernel.reshape(BOND, UNITS, UNITS)
    w2r = w3.transpose(0, 2, 1)
    bt = edge_bias.reshape(UNITS, UNITS).T
    rm4 = (eye4[:, None, None, :, None] * eye16[None, :, :, None, None]
           * jnp.ones((1, 1, 1, 1, UNITS), jnp.float32)
           ).reshape(4 * BOND, 4 * BOND * UNITS).astype(jnp.bfloat16)
    w24 = jnp.einsum('bvu,st->bsvtu', w2r, eye4).reshape(
        4 * BOND * UNITS, 4 * UNITS)
    bt4 = jnp.kron(eye4, bt)

    wz, wr, wh = (gru_kernel[:, :UNITS], gru_kernel[:, UNITS:2 * UNITS],
                  gru_kernel[:, 2 * UNITS:])
    uz, ur, uh = (gru_recurrent_kernel[:, :UNITS],
                  gru_recurrent_kernel[:, UNITS:2 * UNITS],
                  gru_recurrent_kernel[:, 2 * UNITS:])
    wz4, wr4, wh4 = (jnp.kron(eye4, w) for w in (wz, wr, wh))
    uz4, ur4, uh4 = (jnp.kron(eye4, u) for u in (uz, ur, uh))
    bmat = jnp.concatenate(
        [gru_bias.reshape(6, UNITS), jnp.zeros((2, UNITS), jnp.float32)],
        axis=0)
    bmat4 = jnp.tile(bmat, (1, 4))

    idx0, idx1 = pl.pallas_call(
        _idx_kernel,
        out_shape=(jax.ShapeDtypeStruct((EROWS, 128), jnp.int32),
                   jax.ShapeDtypeStruct((EROWS, 128), jnp.int32)),
    )(src2d)
    idx0 = idx0.reshape(16, S_NCH, S_CH, 128)
    idx1 = idx1.reshape(16, S_NCH, S_CH, 128)

    for _ in range(2):
        nbr = _sc_gather(h, dst3)
        nbr4 = nbr.reshape(EPAD4, 4 * UNITS)
        msg4 = _edge_call(bond4, nbr4, rm4, w24, bt4)
        msg = msg4.reshape(EPAD, UNITS)
        agg = _sc_scatter(msg, idx0, idx1, zer)
        agg4 = agg.reshape(NN4, 4 * UNITS)
        h4 = h.reshape(NN4, 4 * UNITS)
        h4 = _gru_call(h4, agg4, wz4, wr4, wh4, uz4, ur4, uh4, bmat4)
        h = h4.reshape(NN, UNITS)
    return h
